# Initial kernel scaffold; baseline (speedup 1.0000x reference)
import jax
import jax.numpy as jnp
from jax.experimental import pallas as pl


def kernel(x, c1_w, c1_b, c1_p, c2_w, c2_b, c2_p, w1, b1, w2, b2, w3, b3):
    raise NotImplementedError("write your pallas kernel here")


# speedup vs baseline: 29.7905x; 29.7905x over previous
import numpy as np
import jax
import jax.numpy as jnp
from jax.experimental import pallas as pl
from jax.experimental.pallas import tpu as pltpu

LANE = 128
BIMG = 32
H1, W1C, K = 32, 32, 5
C0, C1, C2 = 3, 6, 16
HO1 = H1 - K + 1
HP1 = HO1 // 2
HO2 = HP1 - K + 1
HP2 = HO2 // 2


def _np_consts():
    e1a = np.zeros((K, C0, LANE, HO1), np.float32)
    for dw in range(K):
        for ci in range(C0):
            for c in range(HO1):
                e1a[dw, ci, C0 * (c + dw) + ci, c] = 1.0
    e2a = np.zeros((HO1, C1, 2 * LANE), np.float32)
    for c in range(HO1):
        for co in range(C1):
            e2a[c, co, C1 * c + co] = 1.0
    e1b = np.zeros((K, C1, LANE, HO2), np.float32)
    for dw in range(K):
        for ci in range(C1):
            for c in range(HO2):
                e1b[dw, ci, C1 * (c + dw) + ci, c] = 1.0
    e2b = np.zeros((HO2, C2, 2 * LANE), np.float32)
    for c in range(HO2):
        for co in range(C2):
            e2b[c, co, C2 * c + co] = 1.0
    s1e = np.zeros((2 * LANE, LANE), np.float32)
    s1o = np.zeros((2 * LANE, LANE), np.float32)
    for c2 in range(HP1):
        for k in range(C1):
            s1e[C1 * (2 * c2) + k, C1 * c2 + k] = 1.0
            s1o[C1 * (2 * c2 + 1) + k, C1 * c2 + k] = 1.0
    s2e = np.zeros((2 * LANE, LANE), np.float32)
    s2o = np.zeros((2 * LANE, LANE), np.float32)
    for c2 in range(HP2):
        for k in range(C2):
            s2e[C2 * (2 * c2) + k, C2 * c2 + k] = 1.0
            s2o[C2 * (2 * c2 + 1) + k, C2 * c2 + k] = 1.0
    mb1 = np.zeros((LANE, 2 * LANE), np.float32)
    for c in range(HO1):
        for k in range(C1):
            mb1[k, C1 * c + k] = 1.0
    mb2 = np.zeros((LANE, 2 * LANE), np.float32)
    for c in range(HO2):
        for k in range(C2):
            mb2[k, C2 * c + k] = 1.0
    return e1a, e2a, e1b, e2b, s1e, s1o, s2e, s2o, mb1, mb2


_E1A, _E2A, _E1B, _E2B, _S1E, _S1O, _S2E, _S2O, _MB1, _MB2 = _np_consts()


def _lenet_kernel(x_ref, wa_ref, ba_ref, s1e_ref, s1o_ref,
                  wb_ref, bb_ref, s2e_ref, s2o_ref,
                  wf1_ref, bf1_ref, wf2_ref, bf2_ref, wf3_ref, bf3_ref,
                  o_ref):
    b = x_ref.shape[0]
    r1 = b * H1
    xf = x_ref[...].reshape(r1, LANE)

    acc = jnp.dot(xf[0:r1 - 4, :], wa_ref[0],
                  preferred_element_type=jnp.float32)
    for dh in range(1, K):
        acc = acc + jnp.dot(xf[dh:r1 - 4 + dh, :], wa_ref[dh],
                            preferred_element_type=jnp.float32)
    acc = jnp.maximum(acc + ba_ref[...], 0.0)

    accp = jnp.concatenate(
        [acc, jnp.zeros((4, 2 * LANE), jnp.float32)], axis=0)
    a3 = accp.reshape(r1 // 2, 2, 2 * LANE)
    mrow = jnp.maximum(a3[:, 0, :], a3[:, 1, :])
    mrow = mrow.reshape(b, H1 // 2, 2 * LANE)[:, :HP1, :]
    mrow = mrow.reshape(b * HP1, 2 * LANE)
    p1 = jnp.maximum(
        jnp.dot(mrow, s1e_ref[...], preferred_element_type=jnp.float32),
        jnp.dot(mrow, s1o_ref[...], preferred_element_type=jnp.float32))

    r2 = b * HP1
    acc2 = jnp.dot(p1[0:r2 - 4, :], wb_ref[0],
                   preferred_element_type=jnp.float32)
    for dh in range(1, K):
        acc2 = acc2 + jnp.dot(p1[dh:r2 - 4 + dh, :], wb_ref[dh],
                              preferred_element_type=jnp.float32)
    acc2 = jnp.maximum(acc2 + bb_ref[...], 0.0)

    accp2 = jnp.concatenate(
        [acc2, jnp.zeros((4, 2 * LANE), jnp.float32)], axis=0)
    a32 = accp2.reshape(r2 // 2, 2, 2 * LANE)
    mrow2 = jnp.maximum(a32[:, 0, :], a32[:, 1, :])
    mrow2 = mrow2.reshape(b, HP1 // 2, 2 * LANE)[:, :HP2, :]
    mrow2 = mrow2.reshape(b * HP2, 2 * LANE)
    p2 = jnp.maximum(
        jnp.dot(mrow2, s2e_ref[...], preferred_element_type=jnp.float32),
        jnp.dot(mrow2, s2o_ref[...], preferred_element_type=jnp.float32))

    p2r = p2.reshape(b, HP2, LANE)
    h = jnp.dot(p2r[:, 0, :], wf1_ref[0], preferred_element_type=jnp.float32)
    for hh in range(1, HP2):
        h = h + jnp.dot(p2r[:, hh, :], wf1_ref[hh],
                        preferred_element_type=jnp.float32)
    h = jnp.maximum(h + bf1_ref[...], 0.0)
    h = jnp.dot(h, wf2_ref[...], preferred_element_type=jnp.float32)
    h = jnp.maximum(h + bf2_ref[...], 0.0)
    y = jnp.dot(h, wf3_ref[...], preferred_element_type=jnp.float32)
    o_ref[...] = y + bf3_ref[...]


@jax.jit
def kernel(x, c1_w, c1_b, c1_p, c2_w, c2_b, c2_p,
           w1, b1, w2, b2, w3, b3):
    del c1_p, c2_p
    n = x.shape[0]

    wt1 = c1_w.reshape(K, K, LANE, LANE)[:, :, :C0, :C1]
    wa = jnp.einsum('wilc,hwio,com->hlm', _E1A, wt1, _E2A)
    wt2 = c2_w.reshape(K, K, LANE, LANE)[:, :, :C1, :C2]
    wb = jnp.einsum('wilc,hwio,com->hlm', _E1B, wt2, _E2B)
    ba = c1_b @ _MB1
    bb = c2_b @ _MB2
    wf1 = w1.reshape(K, K, LANE, LANE)[:, :, :C2, :].reshape(K, K * C2, LANE)
    wf1 = jnp.pad(wf1, ((0, 0), (0, LANE - K * C2), (0, 0)))

    xp = jnp.transpose(x, (0, 2, 3, 1)).reshape(n, H1, W1C * C0)
    xp = jnp.pad(xp, ((0, 0), (0, 0), (0, LANE - W1C * C0)))

    out = pl.pallas_call(
        _lenet_kernel,
        out_shape=jax.ShapeDtypeStruct((n, LANE), jnp.float32),
        grid=(n // BIMG,),
        in_specs=[
            pl.BlockSpec((BIMG, H1, LANE), lambda i: (i, 0, 0)),
            pl.BlockSpec((K, LANE, 2 * LANE), lambda i: (0, 0, 0)),
            pl.BlockSpec((1, 2 * LANE), lambda i: (0, 0)),
            pl.BlockSpec((2 * LANE, LANE), lambda i: (0, 0)),
            pl.BlockSpec((2 * LANE, LANE), lambda i: (0, 0)),
            pl.BlockSpec((K, LANE, 2 * LANE), lambda i: (0, 0, 0)),
            pl.BlockSpec((1, 2 * LANE), lambda i: (0, 0)),
            pl.BlockSpec((2 * LANE, LANE), lambda i: (0, 0)),
            pl.BlockSpec((2 * LANE, LANE), lambda i: (0, 0)),
            pl.BlockSpec((K, LANE, LANE), lambda i: (0, 0, 0)),
            pl.BlockSpec((1, LANE), lambda i: (0, 0)),
            pl.BlockSpec((LANE, LANE), lambda i: (0, 0)),
            pl.BlockSpec((1, LANE), lambda i: (0, 0)),
            pl.BlockSpec((LANE, LANE), lambda i: (0, 0)),
            pl.BlockSpec((1, LANE), lambda i: (0, 0)),
        ],
        out_specs=pl.BlockSpec((BIMG, LANE), lambda i: (i, 0)),
        compiler_params=pltpu.CompilerParams(
            dimension_semantics=("parallel",),
            vmem_limit_bytes=48 * 1024 * 1024,
        ),
    )(xp, wa, ba, jnp.asarray(_S1E), jnp.asarray(_S1O),
      wb, bb, jnp.asarray(_S2E), jnp.asarray(_S2O),
      wf1, b1, w2, b2, w3, b3)
    return out[:, :10]


# speedup vs baseline: 31.1482x; 1.0456x over previous
import numpy as np
import jax
import jax.numpy as jnp
from jax.experimental import pallas as pl
from jax.experimental.pallas import tpu as pltpu

LANE = 128
BIMG = 32
H1, W1C, K = 32, 32, 5
C0, C1, C2 = 3, 6, 16
HO1 = H1 - K + 1
HP1 = HO1 // 2
HO2 = HP1 - K + 1
HP2 = HO2 // 2


def _np_consts():
    e1a = np.zeros((K, C0, C0 * W1C, HO1), np.float32)
    for dw in range(K):
        for ci in range(C0):
            for c in range(HO1):
                e1a[dw, ci, W1C * ci + c + dw, c] = 1.0
    e2a = np.zeros((HO1, C1, 2 * LANE), np.float32)
    for c in range(HO1):
        for co in range(C1):
            e2a[c, co, C1 * c + co] = 1.0
    e1b = np.zeros((K, C1, LANE, HO2), np.float32)
    for dw in range(K):
        for ci in range(C1):
            for c in range(HO2):
                e1b[dw, ci, C1 * (c + dw) + ci, c] = 1.0
    e2b = np.zeros((HO2, C2, 2 * LANE), np.float32)
    for c in range(HO2):
        for co in range(C2):
            e2b[c, co, C2 * c + co] = 1.0
    s1e = np.zeros((2 * LANE, LANE), np.float32)
    s1o = np.zeros((2 * LANE, LANE), np.float32)
    for c2 in range(HP1):
        for k in range(C1):
            s1e[C1 * (2 * c2) + k, C1 * c2 + k] = 1.0
            s1o[C1 * (2 * c2 + 1) + k, C1 * c2 + k] = 1.0
    s2e = np.zeros((2 * LANE, LANE), np.float32)
    s2o = np.zeros((2 * LANE, LANE), np.float32)
    for c2 in range(HP2):
        for k in range(C2):
            s2e[C2 * (2 * c2) + k, C2 * c2 + k] = 1.0
            s2o[C2 * (2 * c2 + 1) + k, C2 * c2 + k] = 1.0
    mb1 = np.zeros((LANE, 2 * LANE), np.float32)
    for c in range(HO1):
        for k in range(C1):
            mb1[k, C1 * c + k] = 1.0
    mb2 = np.zeros((LANE, 2 * LANE), np.float32)
    for c in range(HO2):
        for k in range(C2):
            mb2[k, C2 * c + k] = 1.0
    return e1a, e2a, e1b, e2b, s1e, s1o, s2e, s2o, mb1, mb2


_E1A, _E2A, _E1B, _E2B, _S1E, _S1O, _S2E, _S2O, _MB1, _MB2 = _np_consts()


def _lenet_kernel(x_ref, wa_ref, ba_ref, s1e_ref, s1o_ref,
                  wb_ref, bb_ref, s2e_ref, s2o_ref,
                  wf1_ref, bf1_ref, wf2_ref, bf2_ref, wf3_ref, bf3_ref,
                  o_ref):
    b = x_ref.shape[0]
    r1 = b * H1
    xr = x_ref[...]
    xcat = jnp.concatenate(
        [xr[:, 0:W1C, :], xr[:, W1C:2 * W1C, :], xr[:, 2 * W1C:3 * W1C, :]],
        axis=2)
    xf = xcat.reshape(r1, C0 * W1C).astype(jnp.bfloat16)

    acc = jnp.dot(xf[0:r1 - 4, :], wa_ref[0],
                  preferred_element_type=jnp.float32)
    for dh in range(1, K):
        acc = acc + jnp.dot(xf[dh:r1 - 4 + dh, :], wa_ref[dh],
                            preferred_element_type=jnp.float32)
    acc = jnp.maximum(acc + ba_ref[...], 0.0)

    accp = jnp.concatenate(
        [acc, jnp.zeros((4, 2 * LANE), jnp.float32)], axis=0)
    a3 = accp.reshape(r1 // 2, 2, 2 * LANE)
    mrow = jnp.maximum(a3[:, 0, :], a3[:, 1, :])
    mrow = mrow.reshape(b, H1 // 2, 2 * LANE)[:, :HP1, :]
    mrow = mrow.reshape(b * HP1, 2 * LANE).astype(jnp.bfloat16)
    p1 = jnp.maximum(
        jnp.dot(mrow, s1e_ref[...], preferred_element_type=jnp.float32),
        jnp.dot(mrow, s1o_ref[...], preferred_element_type=jnp.float32))
    p1 = p1.astype(jnp.bfloat16)

    r2 = b * HP1
    acc2 = jnp.dot(p1[0:r2 - 4, :], wb_ref[0],
                   preferred_element_type=jnp.float32)
    for dh in range(1, K):
        acc2 = acc2 + jnp.dot(p1[dh:r2 - 4 + dh, :], wb_ref[dh],
                              preferred_element_type=jnp.float32)
    acc2 = jnp.maximum(acc2 + bb_ref[...], 0.0)

    accp2 = jnp.concatenate(
        [acc2, jnp.zeros((4, 2 * LANE), jnp.float32)], axis=0)
    a32 = accp2.reshape(r2 // 2, 2, 2 * LANE)
    mrow2 = jnp.maximum(a32[:, 0, :], a32[:, 1, :])
    mrow2 = mrow2.reshape(b, HP1 // 2, 2 * LANE)[:, :HP2, :]
    mrow2 = mrow2.reshape(b * HP2, 2 * LANE).astype(jnp.bfloat16)
    p2 = jnp.maximum(
        jnp.dot(mrow2, s2e_ref[...], preferred_element_type=jnp.float32),
        jnp.dot(mrow2, s2o_ref[...], preferred_element_type=jnp.float32))

    p2r = p2.astype(jnp.bfloat16).reshape(b, HP2, LANE)
    h = jnp.dot(p2r[:, 0, :], wf1_ref[0], preferred_element_type=jnp.float32)
    for hh in range(1, HP2):
        h = h + jnp.dot(p2r[:, hh, :], wf1_ref[hh],
                        preferred_element_type=jnp.float32)
    h = jnp.maximum(h + bf1_ref[...], 0.0).astype(jnp.bfloat16)
    h = jnp.dot(h, wf2_ref[...], preferred_element_type=jnp.float32)
    h = jnp.maximum(h + bf2_ref[...], 0.0).astype(jnp.bfloat16)
    y = jnp.dot(h, wf3_ref[...], preferred_element_type=jnp.float32)
    o_ref[...] = y + bf3_ref[...]


@jax.jit
def kernel(x, c1_w, c1_b, c1_p, c2_w, c2_b, c2_p,
           w1, b1, w2, b2, w3, b3):
    del c1_p, c2_p
    n = x.shape[0]

    bf16 = jnp.bfloat16
    wt1 = c1_w.reshape(K, K, LANE, LANE)[:, :, :C0, :C1]
    wa = jnp.einsum('wilc,hwio,com->hlm', _E1A, wt1, _E2A)
    wt2 = c2_w.reshape(K, K, LANE, LANE)[:, :, :C1, :C2]
    wb = jnp.einsum('wilc,hwio,com->hlm', _E1B, wt2, _E2B)
    ba = c1_b @ _MB1
    bb = c2_b @ _MB2
    wf1 = w1.reshape(K, K, LANE, LANE)[:, :, :C2, :].reshape(K, K * C2, LANE)
    wf1 = jnp.pad(wf1, ((0, 0), (0, LANE - K * C2), (0, 0)))

    xv = x.reshape(n, C0 * H1, W1C)

    out = pl.pallas_call(
        _lenet_kernel,
        out_shape=jax.ShapeDtypeStruct((n, LANE), jnp.float32),
        grid=(n // BIMG,),
        in_specs=[
            pl.BlockSpec((BIMG, C0 * H1, W1C), lambda i: (i, 0, 0)),
            pl.BlockSpec((K, C0 * W1C, 2 * LANE), lambda i: (0, 0, 0)),
            pl.BlockSpec((1, 2 * LANE), lambda i: (0, 0)),
            pl.BlockSpec((2 * LANE, LANE), lambda i: (0, 0)),
            pl.BlockSpec((2 * LANE, LANE), lambda i: (0, 0)),
            pl.BlockSpec((K, LANE, 2 * LANE), lambda i: (0, 0, 0)),
            pl.BlockSpec((1, 2 * LANE), lambda i: (0, 0)),
            pl.BlockSpec((2 * LANE, LANE), lambda i: (0, 0)),
            pl.BlockSpec((2 * LANE, LANE), lambda i: (0, 0)),
            pl.BlockSpec((K, LANE, LANE), lambda i: (0, 0, 0)),
            pl.BlockSpec((1, LANE), lambda i: (0, 0)),
            pl.BlockSpec((LANE, LANE), lambda i: (0, 0)),
            pl.BlockSpec((1, LANE), lambda i: (0, 0)),
            pl.BlockSpec((LANE, LANE), lambda i: (0, 0)),
            pl.BlockSpec((1, LANE), lambda i: (0, 0)),
        ],
        out_specs=pl.BlockSpec((BIMG, LANE), lambda i: (i, 0)),
        compiler_params=pltpu.CompilerParams(
            dimension_semantics=("parallel",),
            vmem_limit_bytes=48 * 1024 * 1024,
        ),
    )(xv, wa.astype(bf16), ba,
      jnp.asarray(_S1E, bf16), jnp.asarray(_S1O, bf16),
      wb.astype(bf16), bb,
      jnp.asarray(_S2E, bf16), jnp.asarray(_S2O, bf16),
      wf1.astype(bf16), b1, w2.astype(bf16), b2, w3.astype(bf16), b3)
    return out[:, :10]
